# baseline (speedup 1.0000x reference)
import functools

import jax
import jax.numpy as jnp
from jax import lax
from jax.experimental import pallas as pl
from jax.experimental.pallas import tpu as pltpu
from jax.experimental.pallas import tpu_sc as plsc

N = 10000
E = 320000
D = 128
G = 64
NP = 10240
NW = 32
EC = E // NW
C = 128
ECP = 10240
BPB = 8
NBLK = ECP // (BPB * C)
NCH = ECP // C
HW = 128
RPT = NP // 16
BATCH_PER_TILE = NP // 16
SB = 80
NB = BATCH_PER_TILE // SB
GP = 80

_MESH = dict(mesh=plsc.VectorSubcoreMesh(core_axis_name="c", subcore_axis_name="s"))


def _wid():
    return lax.axis_index("s") * 2 + lax.axis_index("c")


@functools.partial(
    pl.kernel,
    out_type=[
        jax.ShapeDtypeStruct((2, NP, HW), jnp.float32),
        jax.ShapeDtypeStruct((GP, HW), jnp.float32),
    ],
    scratch_types=[
        pltpu.VMEM((NBLK, BPB, C), jnp.int32),
        pltpu.VMEM((NB, SB), jnp.int32),
        pltpu.VMEM((C, HW), jnp.float32),
        pltpu.VMEM_SHARED((NP, HW), jnp.float32),
        pltpu.VMEM_SHARED((GP, HW), jnp.float32),
    ],
    **_MESH,
)
def _sc_prep(dst_hbm, batch_hbm, ones_hbm, zd_hbm, zc_hbm,
             degp_hbm, cnt_hbm,
             dstv, batv, onesv, deg_acc, cnt_acc):
    c = lax.axis_index("c")
    s = lax.axis_index("s")
    w = _wid()
    pltpu.sync_copy(zd_hbm.at[pl.ds(s * RPT, RPT)], deg_acc.at[pl.ds(s * RPT, RPT)])

    @pl.when(s == 0)
    def _():
        pltpu.sync_copy(zc_hbm, cnt_acc)

    pltpu.sync_copy(ones_hbm, onesv)
    pltpu.sync_copy(dst_hbm.at[w], dstv)

    @pl.when(c == 0)
    def _():
        pltpu.sync_copy(batch_hbm.at[s], batv)

    plsc.subcore_barrier()

    def deg_body(k, _):
        pltpu.sync_copy(onesv, deg_acc.at[dstv.at[k // BPB, k % BPB]], add=True)
        return 0

    lax.fori_loop(0, NCH, deg_body, 0)

    @pl.when(c == 0)
    def _():
        def cnt_body(j, _):
            pltpu.sync_copy(onesv.at[pl.ds(0, SB)], cnt_acc.at[batv.at[j]], add=True)
            return 0
        lax.fori_loop(0, NB, cnt_body, 0)

    plsc.subcore_barrier()
    pltpu.sync_copy(deg_acc.at[pl.ds(s * RPT, RPT)],
                    degp_hbm.at[c, pl.ds(s * RPT, RPT)])

    @pl.when((c == 0) & (s == 0))
    def _():
        pltpu.sync_copy(cnt_acc, cnt_hbm)


@functools.partial(
    pl.kernel,
    out_type=jax.ShapeDtypeStruct((2, NP, D), jnp.float32),
    scratch_types=[
        pltpu.VMEM((BPB, C), jnp.int32),
        pltpu.VMEM((BPB, C), jnp.int32),
        pltpu.VMEM((BPB, C), jnp.int32),
        pltpu.VMEM((BPB, C), jnp.int32),
        pltpu.VMEM((C, D), jnp.float32),
        pltpu.VMEM((C, D), jnp.float32),
        pltpu.VMEM_SHARED((NP, D), jnp.float32),
    ]
    + [pltpu.SemaphoreType.DMA for _ in range(6)],
    **_MESH,
)
def _sc_agg(z_hbm, src_hbm, dst_hbm, zeros_hbm, part_hbm,
            sb0, sb1, db0, db1, g0, g1, acc,
            is0, is1, gs0, gs1, ss0, ss1):
    sstg = (sb0, sb1)
    dstg = (db0, db1)
    gbuf = (g0, g1)
    isem = (is0, is1)
    gsem = (gs0, gs1)
    ssem = (ss0, ss1)
    c = lax.axis_index("c")
    s = lax.axis_index("s")
    w = _wid()
    pltpu.sync_copy(zeros_hbm.at[pl.ds(s * RPT, RPT)], acc.at[pl.ds(s * RPT, RPT)])

    def stage(blk):
        p = blk % 2
        pltpu.async_copy(src_hbm.at[w, blk], sstg[p], isem[p])
        pltpu.async_copy(dst_hbm.at[w, blk], dstg[p], isem[p])

    def stage_wait(blk):
        p = blk % 2
        pltpu.make_async_copy(src_hbm.at[w, blk], sstg[p], isem[p]).wait()
        pltpu.make_async_copy(dst_hbm.at[w, blk], dstg[p], isem[p]).wait()

    stage(0)
    plsc.subcore_barrier()
    stage_wait(0)
    stage(1)

    for k in range(2):
        pltpu.async_copy(z_hbm.at[sstg[0].at[k]], gbuf[k % 2], gsem[k % 2])
    for k in range(NCH):
        blk, jj = divmod(k, BPB)
        p = blk % 2
        b = k % 2
        pltpu.make_async_copy(
            z_hbm.at[sstg[p].at[jj]], gbuf[b], gsem[b]).wait()
        pltpu.async_copy(gbuf[b], acc.at[dstg[p].at[jj]], ssem[b], add=True)
        pltpu.make_async_copy(gbuf[b], acc.at[dstg[p].at[jj]], ssem[b]).wait()
        kn = k + 2
        if kn < NCH:
            blk2, jj2 = divmod(kn, BPB)
            if jj2 == 0:
                stage_wait(blk2)
            pltpu.async_copy(z_hbm.at[sstg[blk2 % 2].at[jj2]], gbuf[b], gsem[b])
        if (k + 1) % BPB == 0 and blk + 2 < NBLK:
            stage(blk + 2)
    plsc.subcore_barrier()
    pltpu.sync_copy(acc.at[pl.ds(s * RPT, RPT)],
                    part_hbm.at[c, pl.ds(s * RPT, RPT)])


@functools.partial(
    pl.kernel,
    out_type=[
        jax.ShapeDtypeStruct((G, D), jnp.float32),
        jax.ShapeDtypeStruct((G, D), jnp.float32),
    ],
    scratch_types=[
        pltpu.VMEM((16,), jnp.int32),
        pltpu.VMEM((16, D), jnp.float32),
        pltpu.VMEM((2, D), jnp.float32),
        pltpu.VMEM((2, D), jnp.float32),
    ],
    **_MESH,
)
def _sc_pool(h_hbm, st_hbm, gmax_hbm, gsum_hbm,
             sbuf, rowbuf, maxo, sumo):
    w = _wid()
    g0 = w * 2
    pltpu.sync_copy(st_hbm.at[w], sbuf)
    desc = sbuf[...]
    for li in range(2):
        start = desc[2 * li]
        cnt = desc[2 * li + 1]
        end = start + cnt
        a16 = (start // 16) * 16
        nch = jnp.where(cnt == 0, 0, (end - a16 + 15) // 16)

        init = tuple([jnp.full((16,), -jnp.inf, jnp.float32)] * 8
                     + [jnp.zeros((16,), jnp.float32)] * 8)

        def body(i, carry, start=start, end=end, a16=a16):
            row0 = a16 + i * 16
            pltpu.sync_copy(h_hbm.at[pl.ds(row0, 16)], rowbuf)
            accs = list(carry)
            for r in range(16):
                row = row0 + r
                valid = ((row >= start) & (row < end)).astype(jnp.float32)
                pen = (valid - 1.0) * 1e30
                for f in range(8):
                    v = rowbuf[r, pl.ds(f * 16, 16)]
                    accs[f] = jnp.maximum(accs[f], v + pen)
                    accs[8 + f] = accs[8 + f] + v * valid
            return tuple(accs)

        res = lax.fori_loop(0, nch, body, init)
        for f in range(8):
            maxo[li, pl.ds(f * 16, 16)] = res[f]
            sumo[li, pl.ds(f * 16, 16)] = res[8 + f]
    pltpu.sync_copy(maxo, gmax_hbm.at[pl.ds(g0, 2)])
    pltpu.sync_copy(sumo, gsum_hbm.at[pl.ds(g0, 2)])


_BLK = 1024
_GRID = NP // _BLK


def _tc_prep_body(degp_ref, cnth_ref, tril_ref, dinv_ref, st_ref, cnt_ref):
    deg = degp_ref[0, :, 0:1] + degp_ref[1, :, 0:1] + 1.0
    dinv_ref[...] = lax.rsqrt(deg)
    cntf = cnth_ref[:, 0:1]
    starts = jnp.dot(tril_ref[...], cntf, preferred_element_type=jnp.float32)
    s2 = starts[:G].reshape(NW, 2)
    c2 = cntf[:G].reshape(NW, 2)
    packed = jnp.stack([s2, c2], axis=2).reshape(NW, 4)
    packed = jnp.concatenate(
        [packed, jnp.zeros((NW, 12), jnp.float32)], axis=1)
    st_ref[...] = packed.astype(jnp.int32)
    cnt_ref[...] = cntf.astype(jnp.int32)


def _tc_prep(degp, cnth, tril):
    return pl.pallas_call(
        _tc_prep_body,
        out_shape=[
            jax.ShapeDtypeStruct((NP, 1), jnp.float32),
            jax.ShapeDtypeStruct((NW, 16), jnp.int32),
            jax.ShapeDtypeStruct((GP, 1), jnp.int32),
        ],
    )(degp, cnth, tril)


def _tc_first_body(x_ref, dinv_ref, w_ref, z_ref):
    z_ref[...] = dinv_ref[...] * jnp.dot(
        x_ref[...], w_ref[...], preferred_element_type=jnp.float32)


def _tc_first(x, dinv, w):
    return pl.pallas_call(
        _tc_first_body,
        grid=(_GRID,),
        in_specs=[
            pl.BlockSpec((_BLK, D), lambda i: (i, 0)),
            pl.BlockSpec((_BLK, 1), lambda i: (i, 0)),
            pl.BlockSpec((D, D), lambda i: (0, 0)),
        ],
        out_specs=pl.BlockSpec((_BLK, D), lambda i: (i, 0)),
        out_shape=jax.ShapeDtypeStruct((NP, D), jnp.float32),
    )(x, dinv, w)


def _tc_mid_body(p_ref, z_ref, dinv_ref, b_ref, w_ref, zn_ref):
    ssum = p_ref[0] + p_ref[1] + z_ref[...]
    h = jnp.tanh(dinv_ref[...] * ssum + b_ref[...])
    zn_ref[...] = dinv_ref[...] * jnp.dot(
        h, w_ref[...], preferred_element_type=jnp.float32)


def _tc_mid(part, z, dinv, b, w):
    return pl.pallas_call(
        _tc_mid_body,
        grid=(_GRID,),
        in_specs=[
            pl.BlockSpec((2, _BLK, D), lambda i: (0, i, 0)),
            pl.BlockSpec((_BLK, D), lambda i: (i, 0)),
            pl.BlockSpec((_BLK, 1), lambda i: (i, 0)),
            pl.BlockSpec((1, D), lambda i: (0, 0)),
            pl.BlockSpec((D, D), lambda i: (0, 0)),
        ],
        out_specs=pl.BlockSpec((_BLK, D), lambda i: (i, 0)),
        out_shape=jax.ShapeDtypeStruct((NP, D), jnp.float32),
    )(part, z, dinv, b, w)


def _tc_last_body(p_ref, z_ref, dinv_ref, b_ref, h_ref):
    ssum = p_ref[0] + p_ref[1] + z_ref[...]
    h_ref[...] = jnp.tanh(dinv_ref[...] * ssum + b_ref[...])


def _tc_last(part, z, dinv, b):
    return pl.pallas_call(
        _tc_last_body,
        grid=(_GRID,),
        in_specs=[
            pl.BlockSpec((2, _BLK, D), lambda i: (0, i, 0)),
            pl.BlockSpec((_BLK, D), lambda i: (i, 0)),
            pl.BlockSpec((_BLK, 1), lambda i: (i, 0)),
            pl.BlockSpec((1, D), lambda i: (0, 0)),
        ],
        out_specs=pl.BlockSpec((_BLK, D), lambda i: (i, 0)),
        out_shape=jax.ShapeDtypeStruct((NP, D), jnp.float32),
    )(part, z, dinv, b)


def _tc_final_body(gmax_ref, gsum_ref, cnt_ref, wo_ref, bo_ref,
                   out_ref, pooled_ref):
    cnt = jnp.maximum(cnt_ref[...], 1.0)
    pooled = jnp.concatenate([gmax_ref[...], gsum_ref[...] / cnt], axis=1)
    pooled_ref[...] = pooled
    out_ref[...] = jnp.dot(pooled, wo_ref[...],
                           preferred_element_type=jnp.float32) + bo_ref[...]


def _tc_final(gmax, gsum, cnt, wo, bo):
    return pl.pallas_call(
        _tc_final_body,
        out_shape=[
            jax.ShapeDtypeStruct((G, 1), jnp.float32),
            jax.ShapeDtypeStruct((G, 2 * D), jnp.float32),
        ],
    )(gmax, gsum, cnt, wo, bo)


def kernel(x, edge_index, batch_index, W_in, b_in, W1, b1, W2, b2, W3, b3,
           W_out, b_out):
    src_e = edge_index[0].astype(jnp.int32).reshape(NW, EC)
    dst_e = edge_index[1].astype(jnp.int32).reshape(NW, EC)
    src_r = jnp.pad(src_e, ((0, 0), (0, ECP - EC))).reshape(NW, NBLK, BPB, C)
    dst_r = jnp.pad(dst_e, ((0, 0), (0, ECP - EC)),
                    constant_values=N).reshape(NW, NBLK, BPB, C)
    batch_p = jnp.concatenate(
        [batch_index.astype(jnp.int32),
         jnp.full((NP - N,), G, jnp.int32)]).reshape(16, NB, SB)
    x_p = jnp.pad(x, ((0, NP - N), (0, 0)))

    ones_h = jnp.ones((C, HW), jnp.float32)
    zeros_deg = jnp.zeros((NP, HW), jnp.float32)
    zeros_cnt = jnp.zeros((GP, HW), jnp.float32)
    zeros_big = jnp.zeros((NP, D), jnp.float32)
    tril = jnp.tril(jnp.ones((GP, GP), jnp.float32), -1)

    degp, cnth = _sc_prep(dst_r, batch_p, ones_h, zeros_deg, zeros_cnt)
    dinv, starts_i, cnt_i = _tc_prep(degp, cnth, tril)

    z = _tc_first(x_p, dinv, W_in)
    for W_next, b_cur in ((W1, b_in), (W2, b1), (W3, b2)):
        part = _sc_agg(z, src_r, dst_r, zeros_big)
        z = _tc_mid(part, z, dinv, b_cur.reshape(1, D), W_next)
    part = _sc_agg(z, src_r, dst_r, zeros_big)
    h4 = _tc_last(part, z, dinv, b3.reshape(1, D))

    gmax, gsum = _sc_pool(h4, starts_i)
    out, pooled = _tc_final(gmax, gsum, cnt_i[:G].astype(jnp.float32),
                            W_out, b_out.reshape(1, 1))
    return (out, pooled)


# speedup vs baseline: 1.0047x; 1.0047x over previous
import functools

import jax
import jax.numpy as jnp
from jax import lax
from jax.experimental import pallas as pl
from jax.experimental.pallas import tpu as pltpu
from jax.experimental.pallas import tpu_sc as plsc

N = 10000
E = 320000
D = 128
G = 64
NP = 10240
NW = 32
EC = E // NW
C = 128
ECP = 10240
BPB = 8
NBLK = ECP // (BPB * C)
NCH = ECP // C
HW = 128
RPT = NP // 16
BATCH_PER_TILE = NP // 16
SB = 80
NB = BATCH_PER_TILE // SB
GP = 80

_MESH = dict(mesh=plsc.VectorSubcoreMesh(core_axis_name="c", subcore_axis_name="s"))


def _wid():
    return lax.axis_index("s") * 2 + lax.axis_index("c")


@functools.partial(
    pl.kernel,
    out_type=[
        jax.ShapeDtypeStruct((2, NP, HW), jnp.float32),
        jax.ShapeDtypeStruct((GP, HW), jnp.float32),
    ],
    scratch_types=[
        pltpu.VMEM((NBLK, BPB, C), jnp.int32),
        pltpu.VMEM((NB, SB), jnp.int32),
        pltpu.VMEM((C, HW), jnp.float32),
        pltpu.VMEM_SHARED((NP, HW), jnp.float32),
        pltpu.VMEM_SHARED((GP, HW), jnp.float32),
    ],
    **_MESH,
)
def _sc_prep(dst_hbm, batch_hbm, ones_hbm, zd_hbm, zc_hbm,
             degp_hbm, cnt_hbm,
             dstv, batv, onesv, deg_acc, cnt_acc):
    c = lax.axis_index("c")
    s = lax.axis_index("s")
    w = _wid()
    pltpu.sync_copy(zd_hbm.at[pl.ds(s * RPT, RPT)], deg_acc.at[pl.ds(s * RPT, RPT)])

    @pl.when(s == 0)
    def _():
        pltpu.sync_copy(zc_hbm, cnt_acc)

    pltpu.sync_copy(ones_hbm, onesv)
    pltpu.sync_copy(dst_hbm.at[w], dstv)

    @pl.when(c == 0)
    def _():
        pltpu.sync_copy(batch_hbm.at[s], batv)

    plsc.subcore_barrier()

    def deg_body(k, _):
        pltpu.sync_copy(onesv, deg_acc.at[dstv.at[k // BPB, k % BPB]], add=True)
        return 0

    lax.fori_loop(0, NCH, deg_body, 0)

    @pl.when(c == 0)
    def _():
        def cnt_body(j, _):
            pltpu.sync_copy(onesv.at[pl.ds(0, SB)], cnt_acc.at[batv.at[j]], add=True)
            return 0
        lax.fori_loop(0, NB, cnt_body, 0)

    plsc.subcore_barrier()
    pltpu.sync_copy(deg_acc.at[pl.ds(s * RPT, RPT)],
                    degp_hbm.at[c, pl.ds(s * RPT, RPT)])

    @pl.when((c == 0) & (s == 0))
    def _():
        pltpu.sync_copy(cnt_acc, cnt_hbm)


@functools.partial(
    pl.kernel,
    out_type=jax.ShapeDtypeStruct((2, NP, D), jnp.float32),
    scratch_types=[
        pltpu.VMEM((BPB, C), jnp.int32),
        pltpu.VMEM((BPB, C), jnp.int32),
        pltpu.VMEM((BPB, C), jnp.int32),
        pltpu.VMEM((BPB, C), jnp.int32),
        pltpu.VMEM((C, D), jnp.float32),
        pltpu.VMEM((C, D), jnp.float32),
        pltpu.VMEM_SHARED((NP, D), jnp.float32),
    ]
    + [pltpu.SemaphoreType.DMA for _ in range(6)],
    **_MESH,
)
def _sc_agg(z_hbm, src_hbm, dst_hbm, zeros_hbm, part_hbm,
            sb0, sb1, db0, db1, g0, g1, acc,
            is0, is1, gs0, gs1, ss0, ss1):
    sstg = (sb0, sb1)
    dstg = (db0, db1)
    gbuf = (g0, g1)
    isem = (is0, is1)
    gsem = (gs0, gs1)
    ssem = (ss0, ss1)
    c = lax.axis_index("c")
    s = lax.axis_index("s")
    w = _wid()
    pltpu.sync_copy(zeros_hbm.at[pl.ds(s * RPT, RPT)], acc.at[pl.ds(s * RPT, RPT)])

    def stage(blk, p):
        pltpu.async_copy(src_hbm.at[w, blk], sstg[p], isem[p])
        pltpu.async_copy(dst_hbm.at[w, blk], dstg[p], isem[p])

    def stage_wait(blk, p):
        pltpu.make_async_copy(src_hbm.at[w, blk], sstg[p], isem[p]).wait()
        pltpu.make_async_copy(dst_hbm.at[w, blk], dstg[p], isem[p]).wait()

    stage(0, 0)
    stage_wait(0, 0)
    stage(1, 1)
    for k in range(2):
        pltpu.async_copy(z_hbm.at[sstg[0].at[k]], gbuf[k % 2], gsem[k % 2])
    plsc.subcore_barrier()

    def chunk(p, jj, b, nxt):
        pltpu.make_async_copy(
            z_hbm.at[sstg[p].at[jj]], gbuf[b], gsem[b]).wait()
        pltpu.async_copy(gbuf[b], acc.at[dstg[p].at[jj]], ssem[b], add=True)
        pltpu.make_async_copy(gbuf[b], acc.at[dstg[p].at[jj]], ssem[b]).wait()
        if nxt is not None:
            p2, jj2 = nxt
            pltpu.async_copy(z_hbm.at[sstg[p2].at[jj2]], gbuf[b], gsem[b])

    def super_body(u, last):
        b0 = 2 * u
        b1 = 2 * u + 1
        for i in range(BPB):
            if i == 6:
                stage_wait(b1, 1)
            chunk(0, i, i % 2, (0, i + 2) if i < 6 else (1, i - 6))
        if not last:
            stage(b0 + 2, 0)
        for i in range(BPB):
            if not last:
                if i == 6:
                    stage_wait(b0 + 2, 0)
                chunk(1, i, i % 2, (1, i + 2) if i < 6 else (0, i - 6))
            else:
                chunk(1, i, i % 2, (1, i + 2) if i < 6 else None)
        if not last:
            stage(b1 + 2, 1)

    def body(u, _):
        super_body(u, last=False)
        return 0

    lax.fori_loop(0, NBLK // 2 - 1, body, 0)
    super_body(NBLK // 2 - 1, last=True)
    plsc.subcore_barrier()
    pltpu.sync_copy(acc.at[pl.ds(s * RPT, RPT)],
                    part_hbm.at[c, pl.ds(s * RPT, RPT)])


@functools.partial(
    pl.kernel,
    out_type=[
        jax.ShapeDtypeStruct((G, D), jnp.float32),
        jax.ShapeDtypeStruct((G, D), jnp.float32),
    ],
    scratch_types=[
        pltpu.VMEM((16,), jnp.int32),
        pltpu.VMEM((16, D), jnp.float32),
        pltpu.VMEM((2, D), jnp.float32),
        pltpu.VMEM((2, D), jnp.float32),
    ],
    **_MESH,
)
def _sc_pool(h_hbm, st_hbm, gmax_hbm, gsum_hbm,
             sbuf, rowbuf, maxo, sumo):
    w = _wid()
    g0 = w * 2
    pltpu.sync_copy(st_hbm.at[w], sbuf)
    desc = sbuf[...]
    for li in range(2):
        start = desc[2 * li]
        cnt = desc[2 * li + 1]
        end = start + cnt
        a16 = (start // 16) * 16
        nch = jnp.where(cnt == 0, 0, (end - a16 + 15) // 16)

        init = tuple([jnp.full((16,), -jnp.inf, jnp.float32)] * 8
                     + [jnp.zeros((16,), jnp.float32)] * 8)

        def body(i, carry, start=start, end=end, a16=a16):
            row0 = a16 + i * 16
            pltpu.sync_copy(h_hbm.at[pl.ds(row0, 16)], rowbuf)
            accs = list(carry)
            for r in range(16):
                row = row0 + r
                valid = ((row >= start) & (row < end)).astype(jnp.float32)
                pen = (valid - 1.0) * 1e30
                for f in range(8):
                    v = rowbuf[r, pl.ds(f * 16, 16)]
                    accs[f] = jnp.maximum(accs[f], v + pen)
                    accs[8 + f] = accs[8 + f] + v * valid
            return tuple(accs)

        res = lax.fori_loop(0, nch, body, init)
        for f in range(8):
            maxo[li, pl.ds(f * 16, 16)] = res[f]
            sumo[li, pl.ds(f * 16, 16)] = res[8 + f]
    pltpu.sync_copy(maxo, gmax_hbm.at[pl.ds(g0, 2)])
    pltpu.sync_copy(sumo, gsum_hbm.at[pl.ds(g0, 2)])


_BLK = 1024
_GRID = NP // _BLK


def _tc_prep_body(degp_ref, cnth_ref, tril_ref, dinv_ref, st_ref, cnt_ref):
    deg = degp_ref[0, :, 0:1] + degp_ref[1, :, 0:1] + 1.0
    dinv_ref[...] = lax.rsqrt(deg)
    cntf = cnth_ref[:, 0:1]
    starts = jnp.dot(tril_ref[...], cntf, preferred_element_type=jnp.float32)
    s2 = starts[:G].reshape(NW, 2)
    c2 = cntf[:G].reshape(NW, 2)
    packed = jnp.stack([s2, c2], axis=2).reshape(NW, 4)
    packed = jnp.concatenate(
        [packed, jnp.zeros((NW, 12), jnp.float32)], axis=1)
    st_ref[...] = packed.astype(jnp.int32)
    cnt_ref[...] = cntf.astype(jnp.int32)


def _tc_prep(degp, cnth, tril):
    return pl.pallas_call(
        _tc_prep_body,
        out_shape=[
            jax.ShapeDtypeStruct((NP, 1), jnp.float32),
            jax.ShapeDtypeStruct((NW, 16), jnp.int32),
            jax.ShapeDtypeStruct((GP, 1), jnp.int32),
        ],
    )(degp, cnth, tril)


def _tc_first_body(x_ref, dinv_ref, w_ref, z_ref):
    z_ref[...] = dinv_ref[...] * jnp.dot(
        x_ref[...], w_ref[...], preferred_element_type=jnp.float32)


def _tc_first(x, dinv, w):
    return pl.pallas_call(
        _tc_first_body,
        grid=(_GRID,),
        in_specs=[
            pl.BlockSpec((_BLK, D), lambda i: (i, 0)),
            pl.BlockSpec((_BLK, 1), lambda i: (i, 0)),
            pl.BlockSpec((D, D), lambda i: (0, 0)),
        ],
        out_specs=pl.BlockSpec((_BLK, D), lambda i: (i, 0)),
        out_shape=jax.ShapeDtypeStruct((NP, D), jnp.float32),
    )(x, dinv, w)


def _tc_mid_body(p_ref, z_ref, dinv_ref, b_ref, w_ref, zn_ref):
    ssum = p_ref[0] + p_ref[1] + z_ref[...]
    h = jnp.tanh(dinv_ref[...] * ssum + b_ref[...])
    zn_ref[...] = dinv_ref[...] * jnp.dot(
        h, w_ref[...], preferred_element_type=jnp.float32)


def _tc_mid(part, z, dinv, b, w):
    return pl.pallas_call(
        _tc_mid_body,
        grid=(_GRID,),
        in_specs=[
            pl.BlockSpec((2, _BLK, D), lambda i: (0, i, 0)),
            pl.BlockSpec((_BLK, D), lambda i: (i, 0)),
            pl.BlockSpec((_BLK, 1), lambda i: (i, 0)),
            pl.BlockSpec((1, D), lambda i: (0, 0)),
            pl.BlockSpec((D, D), lambda i: (0, 0)),
        ],
        out_specs=pl.BlockSpec((_BLK, D), lambda i: (i, 0)),
        out_shape=jax.ShapeDtypeStruct((NP, D), jnp.float32),
    )(part, z, dinv, b, w)


def _tc_last_body(p_ref, z_ref, dinv_ref, b_ref, h_ref):
    ssum = p_ref[0] + p_ref[1] + z_ref[...]
    h_ref[...] = jnp.tanh(dinv_ref[...] * ssum + b_ref[...])


def _tc_last(part, z, dinv, b):
    return pl.pallas_call(
        _tc_last_body,
        grid=(_GRID,),
        in_specs=[
            pl.BlockSpec((2, _BLK, D), lambda i: (0, i, 0)),
            pl.BlockSpec((_BLK, D), lambda i: (i, 0)),
            pl.BlockSpec((_BLK, 1), lambda i: (i, 0)),
            pl.BlockSpec((1, D), lambda i: (0, 0)),
        ],
        out_specs=pl.BlockSpec((_BLK, D), lambda i: (i, 0)),
        out_shape=jax.ShapeDtypeStruct((NP, D), jnp.float32),
    )(part, z, dinv, b)


def _tc_final_body(gmax_ref, gsum_ref, cnt_ref, wo_ref, bo_ref,
                   out_ref, pooled_ref):
    cnt = jnp.maximum(cnt_ref[...], 1.0)
    pooled = jnp.concatenate([gmax_ref[...], gsum_ref[...] / cnt], axis=1)
    pooled_ref[...] = pooled
    out_ref[...] = jnp.dot(pooled, wo_ref[...],
                           preferred_element_type=jnp.float32) + bo_ref[...]


def _tc_final(gmax, gsum, cnt, wo, bo):
    return pl.pallas_call(
        _tc_final_body,
        out_shape=[
            jax.ShapeDtypeStruct((G, 1), jnp.float32),
            jax.ShapeDtypeStruct((G, 2 * D), jnp.float32),
        ],
    )(gmax, gsum, cnt, wo, bo)


def kernel(x, edge_index, batch_index, W_in, b_in, W1, b1, W2, b2, W3, b3,
           W_out, b_out):
    src_e = edge_index[0].astype(jnp.int32).reshape(NW, EC)
    dst_e = edge_index[1].astype(jnp.int32).reshape(NW, EC)
    src_r = jnp.pad(src_e, ((0, 0), (0, ECP - EC))).reshape(NW, NBLK, BPB, C)
    dst_r = jnp.pad(dst_e, ((0, 0), (0, ECP - EC)),
                    constant_values=N).reshape(NW, NBLK, BPB, C)
    batch_p = jnp.concatenate(
        [batch_index.astype(jnp.int32),
         jnp.full((NP - N,), G, jnp.int32)]).reshape(16, NB, SB)
    x_p = jnp.pad(x, ((0, NP - N), (0, 0)))

    ones_h = jnp.ones((C, HW), jnp.float32)
    zeros_deg = jnp.zeros((NP, HW), jnp.float32)
    zeros_cnt = jnp.zeros((GP, HW), jnp.float32)
    zeros_big = jnp.zeros((NP, D), jnp.float32)
    tril = jnp.tril(jnp.ones((GP, GP), jnp.float32), -1)

    degp, cnth = _sc_prep(dst_r, batch_p, ones_h, zeros_deg, zeros_cnt)
    dinv, starts_i, cnt_i = _tc_prep(degp, cnth, tril)

    z = _tc_first(x_p, dinv, W_in)
    for W_next, b_cur in ((W1, b_in), (W2, b1), (W3, b2)):
        part = _sc_agg(z, src_r, dst_r, zeros_big)
        z = _tc_mid(part, z, dinv, b_cur.reshape(1, D), W_next)
    part = _sc_agg(z, src_r, dst_r, zeros_big)
    h4 = _tc_last(part, z, dinv, b3.reshape(1, D))

    gmax, gsum = _sc_pool(h4, starts_i)
    out, pooled = _tc_final(gmax, gsum, cnt_i[:G].astype(jnp.float32),
                            W_out, b_out.reshape(1, 1))
    return (out, pooled)


# speedup vs baseline: 2.7853x; 2.7723x over previous
import functools

import jax
import jax.numpy as jnp
from jax import lax
from jax.experimental import pallas as pl
from jax.experimental.pallas import tpu as pltpu
from jax.experimental.pallas import tpu_sc as plsc

N = 10000
E = 320000
D = 128
G = 64
NP = 10240
NW = 32
EC = E // NW
C = 128
ECP = 10240
BPB = 8
NBLK = ECP // (BPB * C)
NCH = ECP // C
HW = 128
RPT = NP // 16
BATCH_PER_TILE = NP // 16
SB = 80
NB = BATCH_PER_TILE // SB
GP = 80

_MESH = dict(mesh=plsc.VectorSubcoreMesh(core_axis_name="c", subcore_axis_name="s"))


def _wid():
    return lax.axis_index("s") * 2 + lax.axis_index("c")


@functools.partial(
    pl.kernel,
    out_type=[
        jax.ShapeDtypeStruct((2, NP, HW), jnp.float32),
        jax.ShapeDtypeStruct((GP, HW), jnp.float32),
    ],
    scratch_types=[
        pltpu.VMEM((NBLK, BPB, C), jnp.int32),
        pltpu.VMEM((NB, SB), jnp.int32),
        pltpu.VMEM((C, HW), jnp.float32),
        pltpu.VMEM_SHARED((NP, HW), jnp.float32),
        pltpu.VMEM_SHARED((GP, HW), jnp.float32),
    ],
    **_MESH,
)
def _sc_prep(dst_hbm, batch_hbm, ones_hbm, zd_hbm, zc_hbm,
             degp_hbm, cnt_hbm,
             dstv, batv, onesv, deg_acc, cnt_acc):
    c = lax.axis_index("c")
    s = lax.axis_index("s")
    w = _wid()
    pltpu.sync_copy(zd_hbm.at[pl.ds(s * RPT, RPT)], deg_acc.at[pl.ds(s * RPT, RPT)])

    @pl.when(s == 0)
    def _():
        pltpu.sync_copy(zc_hbm, cnt_acc)

    pltpu.sync_copy(ones_hbm, onesv)
    pltpu.sync_copy(dst_hbm.at[w], dstv)

    @pl.when(c == 0)
    def _():
        pltpu.sync_copy(batch_hbm.at[s], batv)

    plsc.subcore_barrier()

    def deg_body(k, _):
        pltpu.sync_copy(onesv, deg_acc.at[dstv.at[k // BPB, k % BPB]], add=True)
        return 0

    lax.fori_loop(0, NCH, deg_body, 0)

    @pl.when(c == 0)
    def _():
        def cnt_body(j, _):
            pltpu.sync_copy(onesv.at[pl.ds(0, SB)], cnt_acc.at[batv.at[j]], add=True)
            return 0
        lax.fori_loop(0, NB, cnt_body, 0)

    plsc.subcore_barrier()
    pltpu.sync_copy(deg_acc.at[pl.ds(s * RPT, RPT)],
                    degp_hbm.at[c, pl.ds(s * RPT, RPT)])

    @pl.when((c == 0) & (s == 0))
    def _():
        pltpu.sync_copy(cnt_acc, cnt_hbm)


@functools.partial(
    pl.kernel,
    out_type=jax.ShapeDtypeStruct((2, NP, D), jnp.float32),
    scratch_types=[
        pltpu.VMEM((BPB, C), jnp.int32),
        pltpu.VMEM((BPB, C), jnp.int32),
        pltpu.VMEM((BPB, C), jnp.int32),
        pltpu.VMEM((BPB, C), jnp.int32),
        pltpu.VMEM((C, D), jnp.float32),
        pltpu.VMEM((C, D), jnp.float32),
        pltpu.VMEM_SHARED((NP, D), jnp.float32),
    ]
    + [pltpu.SemaphoreType.DMA for _ in range(6)],
    **_MESH,
)
def _sc_agg(z_hbm, src_hbm, dst_hbm, zeros_hbm, part_hbm,
            sb0, sb1, db0, db1, g0, g1, acc,
            is0, is1, gs0, gs1, ss0, ss1):
    sstg = (sb0, sb1)
    dstg = (db0, db1)
    gbuf = (g0, g1)
    isem = (is0, is1)
    gsem = (gs0, gs1)
    ssem = (ss0, ss1)
    c = lax.axis_index("c")
    s = lax.axis_index("s")
    w = _wid()
    pltpu.sync_copy(zeros_hbm.at[pl.ds(s * RPT, RPT)], acc.at[pl.ds(s * RPT, RPT)])

    def stage(blk, p):
        pltpu.async_copy(src_hbm.at[w, blk], sstg[p], isem[p])
        pltpu.async_copy(dst_hbm.at[w, blk], dstg[p], isem[p])

    def stage_wait(blk, p):
        pltpu.make_async_copy(src_hbm.at[w, blk], sstg[p], isem[p]).wait()
        pltpu.make_async_copy(dst_hbm.at[w, blk], dstg[p], isem[p]).wait()

    stage(0, 0)
    stage_wait(0, 0)
    stage(1, 1)
    for k in range(2):
        pltpu.async_copy(z_hbm.at[sstg[0].at[k]], gbuf[k % 2], gsem[k % 2])
    plsc.subcore_barrier()

    def chunk(p, jj, b, nxt):
        pltpu.make_async_copy(
            z_hbm.at[sstg[p].at[jj]], gbuf[b], gsem[b]).wait()
        pltpu.async_copy(gbuf[b], acc.at[dstg[p].at[jj]], ssem[b], add=True)
        pltpu.make_async_copy(gbuf[b], acc.at[dstg[p].at[jj]], ssem[b]).wait()
        if nxt is not None:
            p2, jj2 = nxt
            pltpu.async_copy(z_hbm.at[sstg[p2].at[jj2]], gbuf[b], gsem[b])

    def super_body(u, last):
        b0 = 2 * u
        b1 = 2 * u + 1
        for i in range(BPB):
            if i == 6:
                stage_wait(b1, 1)
            chunk(0, i, i % 2, (0, i + 2) if i < 6 else (1, i - 6))
        if not last:
            stage(b0 + 2, 0)
        for i in range(BPB):
            if not last:
                if i == 6:
                    stage_wait(b0 + 2, 0)
                chunk(1, i, i % 2, (1, i + 2) if i < 6 else (0, i - 6))
            else:
                chunk(1, i, i % 2, (1, i + 2) if i < 6 else None)
        if not last:
            stage(b1 + 2, 1)

    def body(u, _):
        super_body(u, last=False)
        return 0

    lax.fori_loop(0, NBLK // 2 - 1, body, 0)
    super_body(NBLK // 2 - 1, last=True)
    plsc.subcore_barrier()
    pltpu.sync_copy(acc.at[pl.ds(s * RPT, RPT)],
                    part_hbm.at[c, pl.ds(s * RPT, RPT)])


@functools.partial(
    pl.kernel,
    out_type=[
        jax.ShapeDtypeStruct((G, D), jnp.float32),
        jax.ShapeDtypeStruct((G, D), jnp.float32),
    ],
    scratch_types=[
        pltpu.VMEM((16,), jnp.int32),
        pltpu.VMEM((16, D), jnp.float32),
        pltpu.VMEM((2, D), jnp.float32),
        pltpu.VMEM((2, D), jnp.float32),
    ],
    **_MESH,
)
def _sc_pool(h_hbm, st_hbm, gmax_hbm, gsum_hbm,
             sbuf, rowbuf, maxo, sumo):
    w = _wid()
    g0 = w * 2
    pltpu.sync_copy(st_hbm.at[w], sbuf)
    desc = sbuf[...]
    for li in range(2):
        start = desc[2 * li]
        cnt = desc[2 * li + 1]
        end = start + cnt
        a16 = (start // 16) * 16
        nch = jnp.where(cnt == 0, 0, (end - a16 + 15) // 16)

        init = tuple([jnp.full((16,), -jnp.inf, jnp.float32)] * 8
                     + [jnp.zeros((16,), jnp.float32)] * 8)

        def body(i, carry, start=start, end=end, a16=a16):
            row0 = a16 + i * 16
            pltpu.sync_copy(h_hbm.at[pl.ds(row0, 16)], rowbuf)
            accs = list(carry)
            for r in range(16):
                row = row0 + r
                valid = ((row >= start) & (row < end)).astype(jnp.float32)
                pen = (valid - 1.0) * 1e30
                for f in range(8):
                    v = rowbuf[r, pl.ds(f * 16, 16)]
                    accs[f] = jnp.maximum(accs[f], v + pen)
                    accs[8 + f] = accs[8 + f] + v * valid
            return tuple(accs)

        res = lax.fori_loop(0, nch, body, init)
        for f in range(8):
            maxo[li, pl.ds(f * 16, 16)] = res[f]
            sumo[li, pl.ds(f * 16, 16)] = res[8 + f]
    pltpu.sync_copy(maxo, gmax_hbm.at[pl.ds(g0, 2)])
    pltpu.sync_copy(sumo, gsum_hbm.at[pl.ds(g0, 2)])


_BLK = 1024
_GRID = NP // _BLK


def _tc_prep_body(degp_ref, cnth_ref, tril_ref, dinv_ref, st_ref, cnt_ref):
    deg = degp_ref[0, :, 0:1] + degp_ref[1, :, 0:1] + 1.0
    dinv_ref[...] = lax.rsqrt(deg)
    cntf = cnth_ref[:, 0:1]
    starts = jnp.dot(tril_ref[...], cntf, preferred_element_type=jnp.float32)
    s2 = starts[:G].reshape(NW, 2)
    c2 = cntf[:G].reshape(NW, 2)
    packed = jnp.stack([s2, c2], axis=2).reshape(NW, 4)
    packed = jnp.concatenate(
        [packed, jnp.zeros((NW, 12), jnp.float32)], axis=1)
    st_ref[...] = packed.astype(jnp.int32)
    cnt_ref[...] = cntf.astype(jnp.int32)


def _tc_prep(degp, cnth, tril):
    return pl.pallas_call(
        _tc_prep_body,
        out_shape=[
            jax.ShapeDtypeStruct((NP, 1), jnp.float32),
            jax.ShapeDtypeStruct((NW, 16), jnp.int32),
            jax.ShapeDtypeStruct((GP, 1), jnp.int32),
        ],
    )(degp, cnth, tril)


def _tc_first_body(x_ref, dinv_ref, w_ref, z_ref):
    z_ref[...] = dinv_ref[...] * jnp.dot(
        x_ref[...], w_ref[...], preferred_element_type=jnp.float32)


def _tc_first(x, dinv, w):
    return pl.pallas_call(
        _tc_first_body,
        grid=(_GRID,),
        in_specs=[
            pl.BlockSpec((_BLK, D), lambda i: (i, 0)),
            pl.BlockSpec((_BLK, 1), lambda i: (i, 0)),
            pl.BlockSpec((D, D), lambda i: (0, 0)),
        ],
        out_specs=pl.BlockSpec((_BLK, D), lambda i: (i, 0)),
        out_shape=jax.ShapeDtypeStruct((NP, D), jnp.float32),
    )(x, dinv, w)


def _tc_mid_body(p_ref, z_ref, dinv_ref, b_ref, w_ref, zn_ref):
    ssum = p_ref[0] + p_ref[1] + z_ref[...]
    h = jnp.tanh(dinv_ref[...] * ssum + b_ref[...])
    zn_ref[...] = dinv_ref[...] * jnp.dot(
        h, w_ref[...], preferred_element_type=jnp.float32)


def _tc_mid(part, z, dinv, b, w):
    return pl.pallas_call(
        _tc_mid_body,
        grid=(_GRID,),
        in_specs=[
            pl.BlockSpec((2, _BLK, D), lambda i: (0, i, 0)),
            pl.BlockSpec((_BLK, D), lambda i: (i, 0)),
            pl.BlockSpec((_BLK, 1), lambda i: (i, 0)),
            pl.BlockSpec((1, D), lambda i: (0, 0)),
            pl.BlockSpec((D, D), lambda i: (0, 0)),
        ],
        out_specs=pl.BlockSpec((_BLK, D), lambda i: (i, 0)),
        out_shape=jax.ShapeDtypeStruct((NP, D), jnp.float32),
    )(part, z, dinv, b, w)


def _tc_last_body(p_ref, z_ref, dinv_ref, b_ref, h_ref):
    ssum = p_ref[0] + p_ref[1] + z_ref[...]
    h_ref[...] = jnp.tanh(dinv_ref[...] * ssum + b_ref[...])


def _tc_last(part, z, dinv, b):
    return pl.pallas_call(
        _tc_last_body,
        grid=(_GRID,),
        in_specs=[
            pl.BlockSpec((2, _BLK, D), lambda i: (0, i, 0)),
            pl.BlockSpec((_BLK, D), lambda i: (i, 0)),
            pl.BlockSpec((_BLK, 1), lambda i: (i, 0)),
            pl.BlockSpec((1, D), lambda i: (0, 0)),
        ],
        out_specs=pl.BlockSpec((_BLK, D), lambda i: (i, 0)),
        out_shape=jax.ShapeDtypeStruct((NP, D), jnp.float32),
    )(part, z, dinv, b)


def _tc_final_body(gmax_ref, gsum_ref, cnt_ref, wo_ref, bo_ref,
                   out_ref, pooled_ref):
    cnt = jnp.maximum(cnt_ref[...], 1.0)
    pooled = jnp.concatenate([gmax_ref[...], gsum_ref[...] / cnt], axis=1)
    pooled_ref[...] = pooled
    out_ref[...] = jnp.dot(pooled, wo_ref[...],
                           preferred_element_type=jnp.float32) + bo_ref[...]


def _tc_final(gmax, gsum, cnt, wo, bo):
    return pl.pallas_call(
        _tc_final_body,
        out_shape=[
            jax.ShapeDtypeStruct((G, 1), jnp.float32),
            jax.ShapeDtypeStruct((G, 2 * D), jnp.float32),
        ],
    )(gmax, gsum, cnt, wo, bo)


def kernel(x, edge_index, batch_index, W_in, b_in, W1, b1, W2, b2, W3, b3,
           W_out, b_out):
    src_e = edge_index[0].astype(jnp.int32).reshape(NW, EC)
    dst_e = edge_index[1].astype(jnp.int32).reshape(NW, EC)
    npad = ECP - EC
    pad_src = jnp.broadcast_to(jnp.arange(npad, dtype=jnp.int32), (NW, npad))
    pad_dst = pad_src + N
    src_r = jnp.concatenate([src_e, pad_src], axis=1).reshape(NW, NBLK, BPB, C)
    dst_r = jnp.concatenate([dst_e, pad_dst], axis=1).reshape(NW, NBLK, BPB, C)
    batch_p = jnp.concatenate(
        [batch_index.astype(jnp.int32),
         jnp.full((NP - N,), G, jnp.int32)]).reshape(16, NB, SB)
    x_p = jnp.pad(x, ((0, NP - N), (0, 0)))

    ones_h = jnp.ones((C, HW), jnp.float32)
    zeros_deg = jnp.zeros((NP, HW), jnp.float32)
    zeros_cnt = jnp.zeros((GP, HW), jnp.float32)
    zeros_big = jnp.zeros((NP, D), jnp.float32)
    tril = jnp.tril(jnp.ones((GP, GP), jnp.float32), -1)

    degp, cnth = _sc_prep(dst_r, batch_p, ones_h, zeros_deg, zeros_cnt)
    dinv, starts_i, cnt_i = _tc_prep(degp, cnth, tril)

    z = _tc_first(x_p, dinv, W_in)
    for W_next, b_cur in ((W1, b_in), (W2, b1), (W3, b2)):
        part = _sc_agg(z, src_r, dst_r, zeros_big)
        z = _tc_mid(part, z, dinv, b_cur.reshape(1, D), W_next)
    part = _sc_agg(z, src_r, dst_r, zeros_big)
    h4 = _tc_last(part, z, dinv, b3.reshape(1, D))

    gmax, gsum = _sc_pool(h4, starts_i)
    out, pooled = _tc_final(gmax, gsum, cnt_i[:G].astype(jnp.float32),
                            W_out, b_out.reshape(1, 1))
    return (out, pooled)


# speedup vs baseline: 2.8096x; 1.0087x over previous
import functools

import jax
import jax.numpy as jnp
from jax import lax
from jax.experimental import pallas as pl
from jax.experimental.pallas import tpu as pltpu
from jax.experimental.pallas import tpu_sc as plsc

N = 10000
E = 320000
D = 128
G = 64
NP = 10240
NW = 32
EC = E // NW
C = 128
ECP = 10240
BPB = 8
NBLK = ECP // (BPB * C)
NCH = ECP // C
HW = 128
RPT = NP // 16
BATCH_PER_TILE = NP // 16
SB = 80
NB = BATCH_PER_TILE // SB
GP = 80

_MESH = dict(mesh=plsc.VectorSubcoreMesh(core_axis_name="c", subcore_axis_name="s"))


def _wid():
    return lax.axis_index("s") * 2 + lax.axis_index("c")


@functools.partial(
    pl.kernel,
    out_type=[
        jax.ShapeDtypeStruct((2, NP, HW), jnp.float32),
        jax.ShapeDtypeStruct((GP, HW), jnp.float32),
    ],
    scratch_types=[
        pltpu.VMEM((NBLK, BPB, C), jnp.int32),
        pltpu.VMEM((NB, SB), jnp.int32),
        pltpu.VMEM((C, HW), jnp.float32),
        pltpu.VMEM_SHARED((NP, HW), jnp.float32),
        pltpu.VMEM_SHARED((GP, HW), jnp.float32),
    ],
    **_MESH,
)
def _sc_prep(dst_hbm, batch_hbm, ones_hbm, zd_hbm, zc_hbm,
             degp_hbm, cnt_hbm,
             dstv, batv, onesv, deg_acc, cnt_acc):
    c = lax.axis_index("c")
    s = lax.axis_index("s")
    w = _wid()
    pltpu.sync_copy(zd_hbm.at[pl.ds(s * RPT, RPT)], deg_acc.at[pl.ds(s * RPT, RPT)])

    @pl.when(s == 0)
    def _():
        pltpu.sync_copy(zc_hbm, cnt_acc)

    pltpu.sync_copy(ones_hbm, onesv)
    pltpu.sync_copy(dst_hbm.at[w], dstv)

    @pl.when(c == 0)
    def _():
        pltpu.sync_copy(batch_hbm.at[s], batv)

    plsc.subcore_barrier()

    def deg_body(k, _):
        pltpu.sync_copy(onesv, deg_acc.at[dstv.at[k // BPB, k % BPB]], add=True)
        return 0

    lax.fori_loop(0, NCH, deg_body, 0)

    @pl.when(c == 0)
    def _():
        def cnt_body(j, _):
            pltpu.sync_copy(onesv.at[pl.ds(0, SB)], cnt_acc.at[batv.at[j]], add=True)
            return 0
        lax.fori_loop(0, NB, cnt_body, 0)

    plsc.subcore_barrier()
    pltpu.sync_copy(deg_acc.at[pl.ds(s * RPT, RPT)],
                    degp_hbm.at[c, pl.ds(s * RPT, RPT)])

    @pl.when((c == 0) & (s == 0))
    def _():
        pltpu.sync_copy(cnt_acc, cnt_hbm)


@functools.partial(
    pl.kernel,
    out_type=jax.ShapeDtypeStruct((2, NP, D), jnp.float32),
    scratch_types=[
        pltpu.VMEM((BPB, C), jnp.int32),
        pltpu.VMEM((BPB, C), jnp.int32),
        pltpu.VMEM((BPB, C), jnp.int32),
        pltpu.VMEM((BPB, C), jnp.int32),
        pltpu.VMEM((C, D), jnp.float32),
        pltpu.VMEM((C, D), jnp.float32),
        pltpu.VMEM_SHARED((NP, D), jnp.float32),
    ]
    + [pltpu.SemaphoreType.DMA for _ in range(6)],
    **_MESH,
)
def _sc_agg(z_hbm, src_hbm, dst_hbm, zeros_hbm, part_hbm,
            sb0, sb1, db0, db1, g0, g1, acc,
            is0, is1, gs0, gs1, ss0, ss1):
    sstg = (sb0, sb1)
    dstg = (db0, db1)
    gbuf = (g0, g1)
    isem = (is0, is1)
    gsem = (gs0, gs1)
    ssem = (ss0, ss1)
    c = lax.axis_index("c")
    s = lax.axis_index("s")
    w = _wid()
    pltpu.sync_copy(zeros_hbm.at[pl.ds(s * RPT, RPT)], acc.at[pl.ds(s * RPT, RPT)])

    def stage(blk, p):
        pltpu.async_copy(src_hbm.at[w, blk], sstg[p], isem[p])
        pltpu.async_copy(dst_hbm.at[w, blk], dstg[p], isem[p])

    def stage_wait(blk, p):
        pltpu.make_async_copy(src_hbm.at[w, blk], sstg[p], isem[p]).wait()
        pltpu.make_async_copy(dst_hbm.at[w, blk], dstg[p], isem[p]).wait()

    stage(0, 0)
    stage_wait(0, 0)
    stage(1, 1)
    for k in range(2):
        pltpu.async_copy(z_hbm.at[sstg[0].at[k]], gbuf[k % 2], gsem[k % 2])
    plsc.subcore_barrier()

    def chunk(p, jj, b, nxt):
        pltpu.make_async_copy(
            z_hbm.at[sstg[p].at[jj]], gbuf[b], gsem[b]).wait()
        pltpu.async_copy(gbuf[b], acc.at[dstg[p].at[jj]], ssem[b], add=True)
        pltpu.make_async_copy(gbuf[b], acc.at[dstg[p].at[jj]], ssem[b]).wait()
        if nxt is not None:
            p2, jj2 = nxt
            pltpu.async_copy(z_hbm.at[sstg[p2].at[jj2]], gbuf[b], gsem[b])

    def super_body(u, last):
        b0 = 2 * u
        b1 = 2 * u + 1
        for i in range(BPB):
            if i == 6:
                stage_wait(b1, 1)
            chunk(0, i, i % 2, (0, i + 2) if i < 6 else (1, i - 6))
        if not last:
            stage(b0 + 2, 0)
        for i in range(BPB):
            if not last:
                if i == 6:
                    stage_wait(b0 + 2, 0)
                chunk(1, i, i % 2, (1, i + 2) if i < 6 else (0, i - 6))
            else:
                chunk(1, i, i % 2, (1, i + 2) if i < 6 else None)
        if not last:
            stage(b1 + 2, 1)

    def body(u, _):
        super_body(u, last=False)
        return 0

    lax.fori_loop(0, NBLK // 2 - 1, body, 0)
    super_body(NBLK // 2 - 1, last=True)
    plsc.subcore_barrier()
    pltpu.sync_copy(acc.at[pl.ds(s * RPT, RPT)],
                    part_hbm.at[c, pl.ds(s * RPT, RPT)])


@functools.partial(
    pl.kernel,
    out_type=[
        jax.ShapeDtypeStruct((G, D), jnp.float32),
        jax.ShapeDtypeStruct((G, D), jnp.float32),
    ],
    scratch_types=[
        pltpu.VMEM((16,), jnp.int32),
        pltpu.VMEM((16, D), jnp.float32),
        pltpu.VMEM((2, D), jnp.float32),
        pltpu.VMEM((2, D), jnp.float32),
    ],
    **_MESH,
)
def _sc_pool(h_hbm, st_hbm, gmax_hbm, gsum_hbm,
             sbuf, rowbuf, maxo, sumo):
    w = _wid()
    g0 = w * 2
    pltpu.sync_copy(st_hbm.at[w], sbuf)
    desc = sbuf[...]
    for li in range(2):
        start = desc[2 * li]
        cnt = desc[2 * li + 1]
        end = start + cnt
        a16 = (start // 16) * 16
        nch = jnp.where(cnt == 0, 0, (end - a16 + 15) // 16)

        init = tuple([jnp.full((16,), -jnp.inf, jnp.float32)] * 8
                     + [jnp.zeros((16,), jnp.float32)] * 8)

        def body(i, carry, start=start, end=end, a16=a16):
            row0 = a16 + i * 16
            pltpu.sync_copy(h_hbm.at[pl.ds(row0, 16)], rowbuf)
            accs = list(carry)
            for r in range(16):
                row = row0 + r
                valid = ((row >= start) & (row < end)).astype(jnp.float32)
                pen = (valid - 1.0) * 1e30
                for f in range(8):
                    v = rowbuf[r, pl.ds(f * 16, 16)]
                    accs[f] = jnp.maximum(accs[f], v + pen)
                    accs[8 + f] = accs[8 + f] + v * valid
            return tuple(accs)

        res = lax.fori_loop(0, nch, body, init)
        for f in range(8):
            maxo[li, pl.ds(f * 16, 16)] = res[f]
            sumo[li, pl.ds(f * 16, 16)] = res[8 + f]
    pltpu.sync_copy(maxo, gmax_hbm.at[pl.ds(g0, 2)])
    pltpu.sync_copy(sumo, gsum_hbm.at[pl.ds(g0, 2)])


_BLK = 1024
_GRID = NP // _BLK


def _tc_prep_body(degp_ref, cnth_ref, tril_ref, y_ref,
                  dinv_ref, st_ref, cnt_ref, z_ref):
    deg = degp_ref[0, :, 0:1] + degp_ref[1, :, 0:1] + 1.0
    dinv = lax.rsqrt(deg)
    dinv_ref[...] = dinv
    z_ref[...] = dinv * y_ref[...]
    cntf = cnth_ref[:, 0:1]
    starts = jnp.dot(tril_ref[...], cntf, preferred_element_type=jnp.float32)
    s2 = starts[:G].reshape(NW, 2)
    c2 = cntf[:G].reshape(NW, 2)
    packed = jnp.stack([s2, c2], axis=2).reshape(NW, 4)
    packed = jnp.concatenate(
        [packed, jnp.zeros((NW, 12), jnp.float32)], axis=1)
    st_ref[...] = packed.astype(jnp.int32)
    cnt_ref[...] = cntf.astype(jnp.int32)


def _tc_prep(degp, cnth, tril, y):
    return pl.pallas_call(
        _tc_prep_body,
        out_shape=[
            jax.ShapeDtypeStruct((NP, 1), jnp.float32),
            jax.ShapeDtypeStruct((NW, 16), jnp.int32),
            jax.ShapeDtypeStruct((GP, 1), jnp.int32),
            jax.ShapeDtypeStruct((NP, D), jnp.float32),
        ],
    )(degp, cnth, tril, y)


def _tc_first_body(x_ref, w_ref, y_ref):
    y_ref[...] = jnp.dot(
        x_ref[...], w_ref[...], preferred_element_type=jnp.float32)


def _tc_first(x, w):
    return pl.pallas_call(
        _tc_first_body,
        grid=(_GRID,),
        in_specs=[
            pl.BlockSpec((_BLK, D), lambda i: (i, 0)),
            pl.BlockSpec((D, D), lambda i: (0, 0)),
        ],
        out_specs=pl.BlockSpec((_BLK, D), lambda i: (i, 0)),
        out_shape=jax.ShapeDtypeStruct((NP, D), jnp.float32),
    )(x, w)


def _tc_mid_body(p_ref, z_ref, dinv_ref, b_ref, w_ref, zn_ref):
    ssum = p_ref[0] + p_ref[1] + z_ref[...]
    h = jnp.tanh(dinv_ref[...] * ssum + b_ref[...])
    zn_ref[...] = dinv_ref[...] * jnp.dot(
        h, w_ref[...], preferred_element_type=jnp.float32)


def _tc_mid(part, z, dinv, b, w):
    return pl.pallas_call(
        _tc_mid_body,
        grid=(_GRID,),
        in_specs=[
            pl.BlockSpec((2, _BLK, D), lambda i: (0, i, 0)),
            pl.BlockSpec((_BLK, D), lambda i: (i, 0)),
            pl.BlockSpec((_BLK, 1), lambda i: (i, 0)),
            pl.BlockSpec((1, D), lambda i: (0, 0)),
            pl.BlockSpec((D, D), lambda i: (0, 0)),
        ],
        out_specs=pl.BlockSpec((_BLK, D), lambda i: (i, 0)),
        out_shape=jax.ShapeDtypeStruct((NP, D), jnp.float32),
    )(part, z, dinv, b, w)


def _tc_last_body(p_ref, z_ref, dinv_ref, b_ref, h_ref):
    ssum = p_ref[0] + p_ref[1] + z_ref[...]
    h_ref[...] = jnp.tanh(dinv_ref[...] * ssum + b_ref[...])


def _tc_last(part, z, dinv, b):
    return pl.pallas_call(
        _tc_last_body,
        grid=(_GRID,),
        in_specs=[
            pl.BlockSpec((2, _BLK, D), lambda i: (0, i, 0)),
            pl.BlockSpec((_BLK, D), lambda i: (i, 0)),
            pl.BlockSpec((_BLK, 1), lambda i: (i, 0)),
            pl.BlockSpec((1, D), lambda i: (0, 0)),
        ],
        out_specs=pl.BlockSpec((_BLK, D), lambda i: (i, 0)),
        out_shape=jax.ShapeDtypeStruct((NP, D), jnp.float32),
    )(part, z, dinv, b)


def _tc_final_body(gmax_ref, gsum_ref, cnt_ref, wo_ref, bo_ref,
                   out_ref, pooled_ref):
    cnt = jnp.maximum(cnt_ref[...], 1.0)
    pooled = jnp.concatenate([gmax_ref[...], gsum_ref[...] / cnt], axis=1)
    pooled_ref[...] = pooled
    out_ref[...] = jnp.dot(pooled, wo_ref[...],
                           preferred_element_type=jnp.float32) + bo_ref[...]


def _tc_final(gmax, gsum, cnt, wo, bo):
    return pl.pallas_call(
        _tc_final_body,
        out_shape=[
            jax.ShapeDtypeStruct((G, 1), jnp.float32),
            jax.ShapeDtypeStruct((G, 2 * D), jnp.float32),
        ],
    )(gmax, gsum, cnt, wo, bo)


def kernel(x, edge_index, batch_index, W_in, b_in, W1, b1, W2, b2, W3, b3,
           W_out, b_out):
    src_e = edge_index[0].astype(jnp.int32).reshape(NW, EC)
    dst_e = edge_index[1].astype(jnp.int32).reshape(NW, EC)
    npad = ECP - EC
    pad_src = jnp.broadcast_to(jnp.arange(npad, dtype=jnp.int32), (NW, npad))
    pad_dst = pad_src + N
    src_r = jnp.concatenate([src_e, pad_src], axis=1).reshape(NW, NBLK, BPB, C)
    dst_r = jnp.concatenate([dst_e, pad_dst], axis=1).reshape(NW, NBLK, BPB, C)
    batch_p = jnp.concatenate(
        [batch_index.astype(jnp.int32),
         jnp.full((NP - N,), G, jnp.int32)]).reshape(16, NB, SB)
    x_p = jnp.pad(x, ((0, NP - N), (0, 0)))

    ones_h = jnp.ones((C, HW), jnp.float32)
    zeros_deg = jnp.zeros((NP, HW), jnp.float32)
    zeros_cnt = jnp.zeros((GP, HW), jnp.float32)
    zeros_big = jnp.zeros((NP, D), jnp.float32)
    tril = jnp.tril(jnp.ones((GP, GP), jnp.float32), -1)

    y1 = _tc_first(x_p, W_in)
    degp, cnth = _sc_prep(dst_r, batch_p, ones_h, zeros_deg, zeros_cnt)
    dinv, starts_i, cnt_i, z = _tc_prep(degp, cnth, tril, y1)

    for W_next, b_cur in ((W1, b_in), (W2, b1), (W3, b2)):
        part = _sc_agg(z, src_r, dst_r, zeros_big)
        z = _tc_mid(part, z, dinv, b_cur.reshape(1, D), W_next)
    part = _sc_agg(z, src_r, dst_r, zeros_big)
    h4 = _tc_last(part, z, dinv, b3.reshape(1, D))

    gmax, gsum = _sc_pool(h4, starts_i)
    out, pooled = _tc_final(gmax, gsum, cnt_i[:G].astype(jnp.float32),
                            W_out, b_out.reshape(1, 1))
    return (out, pooled)
